# baseline (speedup 1.0000x reference)
import jax
import jax.numpy as jnp
from jax import lax
from jax.experimental import pallas as pl
from jax.experimental.pallas import tpu as pltpu

K = 1024
D = 64
B = 16
HW = 1024
T = 512
C = HW // T
NTOK = B * HW
NSTEP = B * C
KC = 256
NKC = K // KC
COMMIT = 0.25


def _vq_body(x_ref, emb_ref, enc_ref, q_ref, loss_ref, perp_ref,
             counts_ref, sse_ref):
    i = pl.program_id(0)

    @pl.when(i == 0)
    def _init():
        counts_ref[...] = jnp.zeros_like(counts_ref)
        sse_ref[0] = 0.0

    x = x_ref[0]
    x2 = jnp.sum(x ** 2, axis=0)
    xd = x + x

    minval = jnp.full((T,), jnp.inf, dtype=jnp.float32)
    idx = jnp.full((T,), K, dtype=jnp.int32)
    for c in range(NKC):
        embc = emb_ref[pl.ds(c * KC, KC), :]
        e2c = jnp.sum(embc ** 2, axis=1)
        xe2 = lax.dot_general(xd, embc, (((0,), (1,)), ((), ())),
                              preferred_element_type=jnp.float32)
        s = (x2[:, None] + e2c[None, :]) - xe2
        cmin = jnp.min(s, axis=1)
        iota = lax.broadcasted_iota(jnp.int32, (T, KC), 1) + (c * KC)
        cidx = jnp.min(jnp.where(s == cmin[:, None], iota, K), axis=1)
        upd = cmin < minval
        minval = jnp.where(upd, cmin, minval)
        idx = jnp.where(upd, cidx, idx)

    qT = jnp.zeros((D, T), dtype=jnp.float32)
    for c in range(NKC):
        iota = lax.broadcasted_iota(jnp.int32, (T, KC), 1) + (c * KC)
        encc = (iota == idx[:, None]).astype(jnp.float32)
        enc_ref[:, pl.ds(c * KC, KC)] = encc
        counts_ref[pl.ds(c * KC, KC)] += jnp.sum(encc, axis=0)
        embc = emb_ref[pl.ds(c * KC, KC), :]
        qT = qT + lax.dot_general(embc, encc, (((0,), (1,)), ((), ())),
                                  preferred_element_type=jnp.float32)

    q_ref[0] = qT
    diff = qT - x
    sse_ref[0] += jnp.sum(diff * diff)

    @pl.when(i == NSTEP - 1)
    def _fini():
        loss_ref[0, 0] = (1.0 + COMMIT) * sse_ref[0] / (NTOK * D)
        avg = counts_ref[...] * (1.0 / NTOK)
        perp_ref[0, 0] = jnp.exp(-jnp.sum(avg * jnp.log(avg + 1e-10)))


def kernel(inputs, embedding):
    xr = inputs.reshape(B, D, HW)
    enc, q, loss, perp = pl.pallas_call(
        _vq_body,
        grid=(NSTEP,),
        in_specs=[
            pl.BlockSpec((1, D, T), lambda i: (i // C, 0, i % C)),
            pl.BlockSpec((K, D), lambda i: (0, 0)),
        ],
        out_specs=[
            pl.BlockSpec((T, K), lambda i: (i, 0)),
            pl.BlockSpec((1, D, T), lambda i: (i // C, 0, i % C)),
            pl.BlockSpec((1, 1), lambda i: (0, 0), memory_space=pltpu.SMEM),
            pl.BlockSpec((1, 1), lambda i: (0, 0), memory_space=pltpu.SMEM),
        ],
        out_shape=[
            jax.ShapeDtypeStruct((NTOK, K), jnp.float32),
            jax.ShapeDtypeStruct((B, D, HW), jnp.float32),
            jax.ShapeDtypeStruct((1, 1), jnp.float32),
            jax.ShapeDtypeStruct((1, 1), jnp.float32),
        ],
        scratch_shapes=[
            pltpu.VMEM((K,), jnp.float32),
            pltpu.SMEM((1,), jnp.float32),
        ],
    )(xr, embedding)
    quantized = q.reshape(B, D, 32, 32)
    return (loss[0, 0], quantized, perp[0, 0], enc)
